# baseline (speedup 1.0000x reference)
import functools

import jax
import jax.numpy as jnp
from jax import lax
from jax.experimental import pallas as pl
from jax.experimental.pallas import tpu as pltpu
from jax.experimental.pallas import tpu_sc as plsc

D = 768
NC, NS = 2, 16
NW = NC * NS
BLK = 512
NCHUNK = 2


def _make_sc_gather(tokens):
    per_w = tokens // NW
    g = 32
    nb = 4
    nch = per_w // g

    mesh = plsc.VectorSubcoreMesh(core_axis_name="c", subcore_axis_name="s")

    @functools.partial(
        pl.kernel,
        mesh=mesh,
        out_type=jax.ShapeDtypeStruct((tokens, D), jnp.float32),
        scratch_types=[
            pltpu.VMEM((per_w,), jnp.int32),
            pltpu.VMEM((nb, g, D), jnp.float32),
            pltpu.SemaphoreType.DMA((nb,)),
            pltpu.SemaphoreType.DMA((nb,)),
        ],
    )
    def sc_gather(ids_hbm, table, out_hbm, idx_v, x_v, in_sem, out_sem):
        wid = lax.axis_index("s") * NC + lax.axis_index("c")
        w0 = wid * per_w
        pltpu.sync_copy(ids_hbm.at[pl.ds(w0, per_w)], idx_v)

        def gather(i, p):
            pltpu.async_copy(table.at[idx_v.at[pl.ds(i * g, g)]],
                             x_v.at[p], in_sem.at[p])

        for p in range(min(nb, nch)):
            gather(p, p)
        for i in range(nch):
            p = i % nb
            pltpu.make_async_copy(table.at[pl.ds(0, g)], x_v.at[p],
                                  in_sem.at[p]).wait()
            pltpu.async_copy(x_v.at[p], out_hbm.at[pl.ds(w0 + i * g, g)],
                             out_sem.at[p])
            if i + nb < nch:
                pltpu.make_async_copy(x_v.at[p], out_hbm.at[pl.ds(0, g)],
                                      out_sem.at[p]).wait()
                gather(i + nb, p)
        for i in range(max(nch - nb, 0), nch):
            p = i % nb
            pltpu.make_async_copy(x_v.at[p], out_hbm.at[pl.ds(0, g)],
                                  out_sem.at[p]).wait()

    return sc_gather


def _ln_math(tok_ref, pos_ref, ttf_ref, seg_ref, g_ref, b_ref, o_ref):
    seg0 = seg_ref[0:1, :]
    segd = seg_ref[1:2, :] - seg0
    x = tok_ref[...] + pos_ref[...] + (seg0 + ttf_ref[...] * segd)
    mean = jnp.mean(x, axis=1, keepdims=True)
    xc = x - mean
    var = jnp.mean(xc * xc, axis=1, keepdims=True)
    inv = lax.rsqrt(var + 1e-12)
    o_ref[...] = xc * inv * g_ref[...] + b_ref[...]


def _tc_body_first(tok_ref, pos_ref, ttf_ref, seg_ref, g_ref, b_ref, o_ref):
    _ln_math(tok_ref, pos_ref, ttf_ref, seg_ref, g_ref, b_ref, o_ref)


def _tc_body_chain(tok_ref, pos_ref, ttf_ref, seg_ref, g_ref, b_ref,
                   prev_ref, o_ref):
    del prev_ref
    _ln_math(tok_ref, pos_ref, ttf_ref, seg_ref, g_ref, b_ref, o_ref)


def _tc_layernorm_chunk(tok_rows, pos_table, ttf, seg_table, gamma, beta,
                        prev, chunk, total_tokens):
    tokens_c = tok_rows.shape[0]
    max_seq = pos_table.shape[0]
    pos_per = max_seq // BLK
    batch_c = tokens_c // max_seq
    blk0 = chunk * (tokens_c // BLK)
    in_specs = [
        pl.BlockSpec((BLK, D), lambda j, b: (b * pos_per + j, 0)),
        pl.BlockSpec((BLK, D), lambda j, b: (j, 0)),
        pl.BlockSpec((BLK, 1), lambda j, b: (b * pos_per + j, 0)),
        pl.BlockSpec((2, D), lambda j, b: (0, 0)),
        pl.BlockSpec((1, D), lambda j, b: (0, 0)),
        pl.BlockSpec((1, D), lambda j, b: (0, 0)),
    ]
    args = [tok_rows, pos_table, ttf, seg_table, gamma, beta]
    if prev is None:
        body = _tc_body_first
        aliases = {}
    else:
        body = _tc_body_chain
        in_specs.append(pl.BlockSpec(memory_space=pl.ANY))
        args.append(prev)
        aliases = {6: 0}
    return pl.pallas_call(
        body,
        grid=(pos_per, batch_c),
        in_specs=in_specs,
        out_specs=pl.BlockSpec((BLK, D),
                               lambda j, b: (blk0 + b * pos_per + j, 0)),
        out_shape=jax.ShapeDtypeStruct((total_tokens, D), jnp.float32),
        input_output_aliases=aliases,
    )(*args)


def kernel(input_ids, token_type_ids, token_table, pos_table, seg_table,
           gamma, beta):
    batch, max_seq = input_ids.shape
    tokens = batch * max_seq
    ids = input_ids.reshape(tokens).astype(jnp.int32)
    ttf = token_type_ids.reshape(tokens, 1).astype(jnp.float32)
    gamma = gamma.reshape(1, D)
    beta = beta.reshape(1, D)

    tokens_c = tokens // NCHUNK
    sc_gather = _make_sc_gather(tokens_c)
    buf = None
    for c in range(NCHUNK):
        sl = slice(c * tokens_c, (c + 1) * tokens_c)
        tok_rows = sc_gather(ids[sl], token_table)
        buf = _tc_layernorm_chunk(tok_rows, pos_table, ttf[sl], seg_table,
                                  gamma, beta, buf, c, tokens)
    return buf.reshape(batch, max_seq, D)


# speedup vs baseline: 1.0615x; 1.0615x over previous
import functools

import jax
import jax.numpy as jnp
from jax import lax
from jax.experimental import pallas as pl
from jax.experimental.pallas import tpu as pltpu
from jax.experimental.pallas import tpu_sc as plsc

D = 768
NC, NS = 2, 16
NW = NC * NS


def _make_sc_gather(tokens):
    per_w = tokens // NW
    g = 32
    nb = 4
    nch = per_w // g

    mesh = plsc.VectorSubcoreMesh(core_axis_name="c", subcore_axis_name="s")

    @functools.partial(
        pl.kernel,
        mesh=mesh,
        out_type=jax.ShapeDtypeStruct((tokens, D), jnp.float32),
        scratch_types=[
            pltpu.VMEM((per_w,), jnp.int32),
            pltpu.VMEM((nb, g, D), jnp.float32),
            pltpu.SemaphoreType.DMA((nb,)),
            pltpu.SemaphoreType.DMA((nb,)),
        ],
    )
    def sc_gather(ids_hbm, table, out_hbm, idx_v, x_v, in_sem, out_sem):
        wid = lax.axis_index("s") * NC + lax.axis_index("c")
        w0 = wid * per_w
        pltpu.sync_copy(ids_hbm.at[pl.ds(w0, per_w)], idx_v)

        def gather(i, p):
            pltpu.async_copy(table.at[idx_v.at[pl.ds(i * g, g)]],
                             x_v.at[p], in_sem.at[p])

        for p in range(nb):
            gather(p, p)
        for i in range(nch):
            p = i % nb
            pltpu.make_async_copy(table.at[pl.ds(0, g)], x_v.at[p],
                                  in_sem.at[p]).wait()
            pltpu.async_copy(x_v.at[p], out_hbm.at[pl.ds(w0 + i * g, g)],
                             out_sem.at[p])
            if i + nb < nch:
                pltpu.make_async_copy(x_v.at[p], out_hbm.at[pl.ds(0, g)],
                                      out_sem.at[p]).wait()
                gather(i + nb, p)
        for i in range(nch - nb, nch):
            p = i % nb
            pltpu.make_async_copy(x_v.at[p], out_hbm.at[pl.ds(0, g)],
                                  out_sem.at[p]).wait()

    return sc_gather


def _tc_ln_body(tok_ref, pos_ref, ttf_ref, seg_ref, g_ref, b_ref, o_ref):
    seg0 = seg_ref[0:1, :]
    segd = seg_ref[1:2, :] - seg0
    x = tok_ref[...] + pos_ref[...] + (seg0 + ttf_ref[...] * segd)
    mean = jnp.mean(x, axis=1, keepdims=True)
    xc = x - mean
    var = jnp.mean(xc * xc, axis=1, keepdims=True)
    inv = lax.rsqrt(var + 1e-12)
    o_ref[...] = xc * inv * g_ref[...] + b_ref[...]


def _tc_layernorm(tok_rows, pos_table, ttf, seg_table, gamma, beta):
    tokens = tok_rows.shape[0]
    max_seq = pos_table.shape[0]
    blk = 1024
    pos_per = max_seq // blk
    batch = tokens // max_seq
    return pl.pallas_call(
        _tc_ln_body,
        grid=(pos_per, batch),
        in_specs=[
            pl.BlockSpec((blk, D), lambda j, b: (b * pos_per + j, 0)),
            pl.BlockSpec((blk, D), lambda j, b: (j, 0)),
            pl.BlockSpec((blk, 1), lambda j, b: (b * pos_per + j, 0)),
            pl.BlockSpec((2, D), lambda j, b: (0, 0)),
            pl.BlockSpec((1, D), lambda j, b: (0, 0)),
            pl.BlockSpec((1, D), lambda j, b: (0, 0)),
        ],
        out_specs=pl.BlockSpec((blk, D), lambda j, b: (b * pos_per + j, 0)),
        out_shape=jax.ShapeDtypeStruct((tokens, D), jnp.float32),
    )(tok_rows, pos_table, ttf, seg_table, gamma, beta)


def kernel(input_ids, token_type_ids, token_table, pos_table, seg_table,
           gamma, beta):
    batch, max_seq = input_ids.shape
    tokens = batch * max_seq
    ids = input_ids.reshape(tokens).astype(jnp.int32)
    ttf = token_type_ids.reshape(tokens, 1).astype(jnp.float32)
    tok_rows = _make_sc_gather(tokens)(ids, token_table)
    out = _tc_layernorm(tok_rows, pos_table, ttf, seg_table,
                        gamma.reshape(1, D), beta.reshape(1, D))
    return out.reshape(batch, max_seq, D)


# speedup vs baseline: 1.0674x; 1.0056x over previous
import functools

import jax
import jax.numpy as jnp
from jax import lax
from jax.experimental import pallas as pl
from jax.experimental.pallas import tpu as pltpu
from jax.experimental.pallas import tpu_sc as plsc

D = 768
NC, NS = 2, 16
NW = NC * NS


def _make_sc_gather(tokens):
    per_w = tokens // NW
    g = 32
    nb = 4
    nch = per_w // g

    mesh = plsc.VectorSubcoreMesh(core_axis_name="c", subcore_axis_name="s")

    @functools.partial(
        pl.kernel,
        mesh=mesh,
        out_type=jax.ShapeDtypeStruct((tokens, D), jnp.float32),
        scratch_types=[
            pltpu.VMEM((per_w,), jnp.int32),
            pltpu.VMEM((nb, g, D), jnp.float32),
            pltpu.SemaphoreType.DMA((nb,)),
            pltpu.SemaphoreType.DMA((nb,)),
        ],
    )
    def sc_gather(ids_hbm, table, out_hbm, idx_v, x_v, in_sem, out_sem):
        wid = lax.axis_index("s") * NC + lax.axis_index("c")
        w0 = wid * per_w
        pltpu.sync_copy(ids_hbm.at[pl.ds(w0, per_w)], idx_v)

        def gather(i, p):
            pltpu.async_copy(table.at[idx_v.at[pl.ds(i * g, g)]],
                             x_v.at[p], in_sem.at[p])

        for p in range(nb):
            gather(p, p)
        for i in range(nch):
            p = i % nb
            pltpu.make_async_copy(table.at[pl.ds(0, g)], x_v.at[p],
                                  in_sem.at[p]).wait()
            pltpu.async_copy(x_v.at[p], out_hbm.at[pl.ds(w0 + i * g, g)],
                             out_sem.at[p])
            if i + nb < nch:
                pltpu.make_async_copy(x_v.at[p], out_hbm.at[pl.ds(0, g)],
                                      out_sem.at[p]).wait()
                gather(i + nb, p)
        for i in range(nch - nb, nch):
            p = i % nb
            pltpu.make_async_copy(x_v.at[p], out_hbm.at[pl.ds(0, g)],
                                  out_sem.at[p]).wait()

    return sc_gather


def _tc_ln_body(tok_ref, pos_ref, ttf_ref, seg_ref, g_ref, b_ref, o_ref):
    seg0 = seg_ref[0:1, :]
    segd = seg_ref[1:2, :] - seg0
    x = tok_ref[...] + pos_ref[...] + (seg0 + ttf_ref[...] * segd)
    mean = jnp.mean(x, axis=1, keepdims=True)
    xc = x - mean
    var = jnp.mean(xc * xc, axis=1, keepdims=True)
    inv = lax.rsqrt(var + 1e-12)
    o_ref[...] = xc * inv * g_ref[...] + b_ref[...]


def _tc_layernorm(tok_rows, pos_table, ttf, seg_table, gamma, beta):
    tokens = tok_rows.shape[0]
    max_seq = pos_table.shape[0]
    blk = 2048
    pos_per = max_seq // blk
    batch = tokens // max_seq
    return pl.pallas_call(
        _tc_ln_body,
        grid=(pos_per, batch),
        in_specs=[
            pl.BlockSpec((blk, D), lambda j, b: (b * pos_per + j, 0)),
            pl.BlockSpec((blk, D), lambda j, b: (j, 0)),
            pl.BlockSpec((blk, 1), lambda j, b: (b * pos_per + j, 0)),
            pl.BlockSpec((2, D), lambda j, b: (0, 0)),
            pl.BlockSpec((1, D), lambda j, b: (0, 0)),
            pl.BlockSpec((1, D), lambda j, b: (0, 0)),
        ],
        out_specs=pl.BlockSpec((blk, D), lambda j, b: (b * pos_per + j, 0)),
        out_shape=jax.ShapeDtypeStruct((tokens, D), jnp.float32),
    )(tok_rows, pos_table, ttf, seg_table, gamma, beta)


def kernel(input_ids, token_type_ids, token_table, pos_table, seg_table,
           gamma, beta):
    batch, max_seq = input_ids.shape
    tokens = batch * max_seq
    ids = input_ids.reshape(tokens).astype(jnp.int32)
    ttf = token_type_ids.reshape(tokens, 1).astype(jnp.float32)
    tok_rows = _make_sc_gather(tokens)(ids, token_table)
    out = _tc_layernorm(tok_rows, pos_table, ttf, seg_table,
                        gamma.reshape(1, D), beta.reshape(1, D))
    return out.reshape(batch, max_seq, D)


# speedup vs baseline: 1.0688x; 1.0013x over previous
import functools

import jax
import jax.numpy as jnp
from jax import lax
from jax.experimental import pallas as pl
from jax.experimental.pallas import tpu as pltpu
from jax.experimental.pallas import tpu_sc as plsc

D = 768
NC, NS = 2, 16
NW = NC * NS


def _make_sc_gather(tokens):
    per_w = tokens // NW
    g = 32
    nb = 4
    nch = per_w // g

    mesh = plsc.VectorSubcoreMesh(core_axis_name="c", subcore_axis_name="s")

    @functools.partial(
        pl.kernel,
        mesh=mesh,
        out_type=jax.ShapeDtypeStruct((tokens, D), jnp.float32),
        scratch_types=[
            pltpu.VMEM((per_w,), jnp.int32),
            pltpu.VMEM((nb, g, D), jnp.float32),
            pltpu.SemaphoreType.DMA((nb,)),
            pltpu.SemaphoreType.DMA((nb,)),
        ],
    )
    def sc_gather(ids_hbm, table, out_hbm, idx_v, x_v, in_sem, out_sem):
        wid = lax.axis_index("s") * NC + lax.axis_index("c")
        w0 = wid * per_w
        pltpu.sync_copy(ids_hbm.at[pl.ds(w0, per_w)], idx_v)

        def gather(i, p):
            pltpu.async_copy(table.at[idx_v.at[pl.ds(i * g, g)]],
                             x_v.at[p], in_sem.at[p])

        for p in range(nb):
            gather(p, p)
        for i in range(nch):
            p = i % nb
            pltpu.make_async_copy(table.at[pl.ds(0, g)], x_v.at[p],
                                  in_sem.at[p]).wait()
            pltpu.async_copy(x_v.at[p], out_hbm.at[pl.ds(w0 + i * g, g)],
                             out_sem.at[p])
            if i + nb < nch:
                pltpu.make_async_copy(x_v.at[p], out_hbm.at[pl.ds(0, g)],
                                      out_sem.at[p]).wait()
                gather(i + nb, p)
        for i in range(nch - nb, nch):
            p = i % nb
            pltpu.make_async_copy(x_v.at[p], out_hbm.at[pl.ds(0, g)],
                                  out_sem.at[p]).wait()

    return sc_gather


def _tc_ln_body(tok_ref, pos_ref, ttf_ref, seg_ref, g_ref, b_ref, o_ref):
    seg0 = seg_ref[0:1, :]
    segd = seg_ref[1:2, :] - seg0
    x = tok_ref[...] + pos_ref[...] + (seg0 + ttf_ref[...] * segd)
    mean = jnp.mean(x, axis=1, keepdims=True)
    xc = x - mean
    var = jnp.mean(xc * xc, axis=1, keepdims=True)
    inv = lax.rsqrt(var + 1e-12)
    o_ref[...] = xc * inv * g_ref[...] + b_ref[...]


def _tc_layernorm(tok_rows, pos_table, ttf, seg_table, gamma, beta):
    tokens = tok_rows.shape[0]
    max_seq = pos_table.shape[0]
    blk = 2048
    pos_per = max_seq // blk
    batch = tokens // max_seq
    return pl.pallas_call(
        _tc_ln_body,
        grid=(pos_per, batch),
        in_specs=[
            pl.BlockSpec((blk, D), lambda j, b: (b * pos_per + j, 0)),
            pl.BlockSpec((blk, D), lambda j, b: (j, 0)),
            pl.BlockSpec((blk, 1), lambda j, b: (b * pos_per + j, 0)),
            pl.BlockSpec((2, D), lambda j, b: (0, 0)),
            pl.BlockSpec((1, D), lambda j, b: (0, 0)),
            pl.BlockSpec((1, D), lambda j, b: (0, 0)),
        ],
        out_specs=pl.BlockSpec((blk, D), lambda j, b: (b * pos_per + j, 0)),
        out_shape=jax.ShapeDtypeStruct((tokens, D), jnp.float32),
        compiler_params=pltpu.CompilerParams(
            dimension_semantics=("parallel", "parallel")),
    )(tok_rows, pos_table, ttf, seg_table, gamma, beta)


def kernel(input_ids, token_type_ids, token_table, pos_table, seg_table,
           gamma, beta):
    batch, max_seq = input_ids.shape
    tokens = batch * max_seq
    ids = input_ids.reshape(tokens).astype(jnp.int32)
    ttf = token_type_ids.reshape(tokens, 1).astype(jnp.float32)
    tok_rows = _make_sc_gather(tokens)(ids, token_table)
    out = _tc_layernorm(tok_rows, pos_table, ttf, seg_table,
                        gamma.reshape(1, D), beta.reshape(1, D))
    return out.reshape(batch, max_seq, D)
